# baseline (speedup 1.0000x reference)
import functools

import jax
import jax.numpy as jnp
from jax import lax
from jax.experimental import pallas as pl
from jax.experimental.pallas import tpu as pltpu
from jax.experimental.pallas import tpu_sc as plsc

_NCHUNKS = 4
_NBUF = 2


def kernel(sequence_tensor, positions):
    batch_size, seq_length, width = sequence_tensor.shape
    nb, npos = positions.shape
    total = nb * npos

    flat_table = sequence_tensor.reshape(batch_size * seq_length, width)

    info = plsc.get_sparse_core_info()
    num_cores = info.num_cores
    num_workers = num_cores * info.num_subcores
    b_per_w = total // num_workers
    chunk = b_per_w // _NCHUNKS

    pos32 = positions.astype(jnp.int32).reshape(num_workers, _NCHUNKS, chunk)

    mesh = plsc.VectorSubcoreMesh(core_axis_name="c", subcore_axis_name="s")

    @functools.partial(
        pl.kernel,
        mesh=mesh,
        out_type=jax.ShapeDtypeStruct((total, width), jnp.float32),
        scratch_types=[
            pltpu.VMEM((_NCHUNKS, chunk), jnp.int32),
            pltpu.VMEM((_NBUF, chunk, width), jnp.float32),
            pltpu.SemaphoreType.DMA,
        ],
    )
    def gather_k(table_hbm, idx_hbm, out_hbm, idx_v, rows_v, gsem):
        wid = lax.axis_index("s") * num_cores + lax.axis_index("c")
        base = wid * b_per_w
        pltpu.sync_copy(idx_hbm.at[wid], idx_v)
        off = (base // npos) * seq_length
        for c in range(_NCHUNKS):
            for i in range(chunk // 16):
                sl = pl.ds(i * 16, 16)
                idx_v[c, sl] = idx_v[c, sl] + off

        gathers = [None] * _NCHUNKS
        gathers[0] = pltpu.async_copy(
            table_hbm.at[idx_v.at[0]], rows_v.at[0], gsem
        )
        for c in range(_NCHUNKS):
            if c + 1 < _NCHUNKS:
                gathers[c + 1] = pltpu.async_copy(
                    table_hbm.at[idx_v.at[c + 1]],
                    rows_v.at[(c + 1) % _NBUF],
                    gsem,
                )
            gathers[c].wait()
            pltpu.sync_copy(
                rows_v.at[c % _NBUF],
                out_hbm.at[pl.ds(base + c * chunk, chunk)],
            )

    return gather_k(flat_table, pos32)


# speedup vs baseline: 1.0332x; 1.0332x over previous
import functools

import jax
import jax.numpy as jnp
from jax import lax
from jax.experimental import pallas as pl
from jax.experimental.pallas import tpu as pltpu
from jax.experimental.pallas import tpu_sc as plsc


def kernel(sequence_tensor, positions):
    batch_size, seq_length, width = sequence_tensor.shape
    nb, npos = positions.shape
    total = nb * npos

    pos32 = positions.astype(jnp.int32)

    info = plsc.get_sparse_core_info()
    num_cores = info.num_cores
    num_workers = num_cores * info.num_subcores
    b_per_w = total // num_workers
    w_per_batch = npos // b_per_w

    mesh = plsc.VectorSubcoreMesh(core_axis_name="c", subcore_axis_name="s")

    @functools.partial(
        pl.kernel,
        mesh=mesh,
        out_type=jax.ShapeDtypeStruct((total, width), jnp.float32),
        scratch_types=[
            pltpu.VMEM((b_per_w,), jnp.int32),
            pltpu.VMEM((b_per_w, width), jnp.float32),
            pltpu.SemaphoreType.DMA,
        ],
    )
    def gather_k(table_hbm, idx_hbm, out_hbm, idx_v, rows_v, sem):
        wid = lax.axis_index("s") * num_cores + lax.axis_index("c")
        b = wid // w_per_batch
        col = (wid % w_per_batch) * b_per_w
        pltpu.sync_copy(idx_hbm.at[b, pl.ds(col, b_per_w)], idx_v)
        pltpu.async_copy(table_hbm.at[b].at[idx_v], rows_v, sem).wait()
        pltpu.sync_copy(rows_v, out_hbm.at[pl.ds(wid * b_per_w, b_per_w)])

    return gather_k(sequence_tensor, pos32)
